# baseline (speedup 1.0000x reference)
import functools

import jax
import jax.numpy as jnp
from jax import lax
from jax.experimental import pallas as pl
from jax.experimental.pallas import tpu as pltpu
from jax.experimental.pallas import tpu_sc as plsc

N_ROWS = 100000
N_COLS = 128
N_SEG = 1024
NC = 2
NS = 16
CPB = N_COLS // NC
CHUNK = 128
GROUP = 256
KPG = GROUP // CHUNK
GPT = 16
SC_ROWS = NS * GPT * GROUP
NBUF = 6
R_TC = 4096
TC_CHUNKS = 8
TC_END = SC_ROWS + TC_CHUNKS * R_TC
EXTRA_OFF = TC_END
N_EXTRA = 12
TAIL_OFF = 99840
REM = 32
REM_OFF = N_ROWS - REM
SEG_PER_TILE = N_SEG // NS

_mesh = plsc.VectorSubcoreMesh(
    core_axis_name="c", subcore_axis_name="s", num_cores=NC, num_subcores=NS
)


@functools.partial(
    pl.kernel,
    out_type=jax.ShapeDtypeStruct((N_SEG, N_COLS), jnp.float32),
    mesh=_mesh,
    scratch_types=[
        pltpu.VMEM((GROUP, CPB), jnp.float32),
        pltpu.VMEM((GROUP, CPB), jnp.float32),
        pltpu.VMEM((GROUP, CPB), jnp.float32),
        pltpu.VMEM((GROUP, CPB), jnp.float32),
        pltpu.VMEM((GROUP, CPB), jnp.float32),
        pltpu.VMEM((GROUP, CPB), jnp.float32),
        pltpu.VMEM((KPG, CHUNK), jnp.int32),
        pltpu.VMEM((KPG, CHUNK), jnp.int32),
        pltpu.VMEM((KPG, CHUNK), jnp.int32),
        pltpu.VMEM((KPG, CHUNK), jnp.int32),
        pltpu.VMEM((KPG, CHUNK), jnp.int32),
        pltpu.VMEM((KPG, CHUNK), jnp.int32),
        pltpu.VMEM((CHUNK, CPB), jnp.float32),
        pltpu.VMEM((CHUNK,), jnp.int32),
        pltpu.VMEM((REM, CPB), jnp.float32),
        pltpu.VMEM((REM,), jnp.int32),
        pltpu.VMEM_SHARED((N_SEG, CPB), jnp.float32),
        pltpu.SemaphoreType.DMA,
        pltpu.SemaphoreType.DMA,
        pltpu.SemaphoreType.DMA,
        pltpu.SemaphoreType.DMA,
        pltpu.SemaphoreType.DMA,
        pltpu.SemaphoreType.DMA,
        pltpu.SemaphoreType.DMA,
        pltpu.SemaphoreType.DMA,
        pltpu.SemaphoreType.DMA,
        pltpu.SemaphoreType.DMA,
        pltpu.SemaphoreType.DMA,
        pltpu.SemaphoreType.DMA,
    ],
    compiler_params=pltpu.CompilerParams(use_tc_tiling_on_sc=False),
)
def _seg_sum_sc(feat_hbm, ids_hbm, out_hbm,
                rows0, rows1, rows2, rows3, rows4, rows5,
                idx0, idx1, idx2, idx3, idx4, idx5,
                rows_t, idx_t, rows_r, idx_r, acc,
                ld0, ld1, ld2, ld3, ld4, ld5,
                sc0, sc1, sc2, sc3, sc4, sc5):
    c = lax.axis_index("c")
    s = lax.axis_index("s")
    col0 = c * CPB
    gstart = s * GPT

    rows = (rows0, rows1, rows2, rows3, rows4, rows5)
    idx = (idx0, idx1, idx2, idx3, idx4, idx5)
    ld = (ld0, ld1, ld2, ld3, ld4, ld5)
    sc = (sc0, sc1, sc2, sc3, sc4, sc5)

    def start_load(g, b):
        off = (gstart + g) * GROUP
        pltpu.async_copy(
            feat_hbm.at[pl.ds(off, GROUP), pl.ds(col0, CPB)], rows[b], ld[b])
        for k in range(KPG):
            pltpu.async_copy(ids_hbm.at[pl.ds(off + k * CHUNK, CHUNK)],
                             idx[b].at[k], ld[b])

    def wait_load(b):
        pltpu.make_async_copy(feat_hbm.at[pl.ds(0, GROUP), pl.ds(0, CPB)],
                              rows[b], ld[b]).wait()
        for k in range(KPG):
            pltpu.make_async_copy(ids_hbm.at[pl.ds(0, CHUNK)],
                                  idx[b].at[k], ld[b]).wait()

    def start_scatters(b):
        for k in range(KPG):
            pltpu.async_copy(rows[b].at[pl.ds(k * CHUNK, CHUNK)],
                             acc.at[idx[b].at[k]], sc[b], add=True)

    def wait_scatters(b):
        for k in range(KPG):
            pltpu.make_async_copy(rows[b].at[pl.ds(k * CHUNK, CHUNK)],
                                  acc.at[idx[b].at[k]], sc[b]).wait()

    zrow = jnp.zeros((16,), jnp.float32)

    def zero_body(r, carry):
        for j in range(CPB // 16):
            rows0[r, pl.ds(j * 16, 16)] = zrow
        return carry

    lax.fori_loop(0, SEG_PER_TILE, zero_body, 0)
    pltpu.sync_copy(rows0.at[pl.ds(0, SEG_PER_TILE)],
                    acc.at[pl.ds(s * SEG_PER_TILE, SEG_PER_TILE)])

    for b in range(NBUF):
        start_load(b, b)
    plsc.subcore_barrier()

    rounds = [list(range(r, min(r + NBUF, GPT))) for r in range(0, GPT, NBUF)]
    for rnd in rounds:
        for g in rnd:
            b = g % NBUF
            wait_load(b)
            start_scatters(b)
        for g in rnd:
            b = g % NBUF
            wait_scatters(b)
            if g + NBUF < GPT:
                start_load(g + NBUF, b)

    @pl.when(s < N_EXTRA)
    def _():
        off = EXTRA_OFF + s * CHUNK
        pltpu.sync_copy(feat_hbm.at[pl.ds(off, CHUNK), pl.ds(col0, CPB)],
                        rows_t)
        pltpu.sync_copy(ids_hbm.at[pl.ds(off, CHUNK)], idx_t)
        pltpu.sync_copy(rows_t, acc.at[idx_t], add=True)

    @pl.when(s == N_EXTRA)
    def _():
        pltpu.sync_copy(feat_hbm.at[pl.ds(TAIL_OFF, CHUNK), pl.ds(col0, CPB)],
                        rows_t)
        pltpu.sync_copy(ids_hbm.at[pl.ds(TAIL_OFF, CHUNK)], idx_t)
        pltpu.sync_copy(rows_t, acc.at[idx_t], add=True)

    @pl.when(s == NS - 1)
    def _():
        pltpu.sync_copy(feat_hbm.at[pl.ds(REM_OFF, REM), pl.ds(col0, CPB)],
                        rows_r)
        pltpu.sync_copy(ids_hbm.at[pl.ds(REM_OFF, REM)], idx_r)
        pltpu.sync_copy(rows_r, acc.at[idx_r], add=True)

    plsc.subcore_barrier()
    pltpu.sync_copy(acc.at[pl.ds(s * SEG_PER_TILE, SEG_PER_TILE)],
                    out_hbm.at[pl.ds(s * SEG_PER_TILE, SEG_PER_TILE),
                               pl.ds(col0, CPB)])


def _tc_body(ids_ref, feat_ref, out_ref):
    i = pl.program_id(0)
    ids = ids_ref[...]
    seg = lax.broadcasted_iota(jnp.int32, (N_SEG, R_TC), 0)
    onehot = (seg == ids[None, :]).astype(jnp.bfloat16)
    fb = feat_ref[...].astype(jnp.bfloat16)
    part = jnp.dot(onehot, fb, preferred_element_type=jnp.float32)

    @pl.when(i == 0)
    def _():
        out_ref[...] = part

    @pl.when(i > 0)
    def _():
        out_ref[...] += part


_seg_sum_tc = pl.pallas_call(
    _tc_body,
    grid=(TC_CHUNKS,),
    in_specs=[
        pl.BlockSpec((R_TC,), lambda i: (SC_ROWS // R_TC + i,)),
        pl.BlockSpec((R_TC, N_COLS), lambda i: (SC_ROWS // R_TC + i, 0)),
    ],
    out_specs=pl.BlockSpec((N_SEG, N_COLS), lambda i: (0, 0)),
    out_shape=jax.ShapeDtypeStruct((N_SEG, N_COLS), jnp.float32),
)


def kernel(feat, segment_ids):
    ids = segment_ids.astype(jnp.int32)
    sc_part = _seg_sum_sc(feat, ids)
    tc_part = _seg_sum_tc(ids, feat)
    return sc_part + tc_part


# speedup vs baseline: 1.0326x; 1.0326x over previous
import functools

import jax
import jax.numpy as jnp
from jax import lax
from jax.experimental import pallas as pl
from jax.experimental.pallas import tpu as pltpu
from jax.experimental.pallas import tpu_sc as plsc

N_ROWS = 100000
N_COLS = 128
N_SEG = 1024
NC = 2
NS = 16
CPB = N_COLS // NC
CHUNK = 128
GROUP = 256
KPG = GROUP // CHUNK
GPT = 16
SC_ROWS = NS * GPT * GROUP
NBUF = 6
R_TC = 2048
TC_CHUNKS = 16
TC_END = SC_ROWS + TC_CHUNKS * R_TC
EXTRA_OFF = TC_END
N_EXTRA = 12
TAIL_OFF = 99840
REM = 32
REM_OFF = N_ROWS - REM
SEG_PER_TILE = N_SEG // NS

_mesh = plsc.VectorSubcoreMesh(
    core_axis_name="c", subcore_axis_name="s", num_cores=NC, num_subcores=NS
)


@functools.partial(
    pl.kernel,
    out_type=jax.ShapeDtypeStruct((N_SEG, N_COLS), jnp.float32),
    mesh=_mesh,
    scratch_types=[
        pltpu.VMEM((GROUP, CPB), jnp.float32),
        pltpu.VMEM((GROUP, CPB), jnp.float32),
        pltpu.VMEM((GROUP, CPB), jnp.float32),
        pltpu.VMEM((GROUP, CPB), jnp.float32),
        pltpu.VMEM((GROUP, CPB), jnp.float32),
        pltpu.VMEM((GROUP, CPB), jnp.float32),
        pltpu.VMEM((KPG, CHUNK), jnp.int32),
        pltpu.VMEM((KPG, CHUNK), jnp.int32),
        pltpu.VMEM((KPG, CHUNK), jnp.int32),
        pltpu.VMEM((KPG, CHUNK), jnp.int32),
        pltpu.VMEM((KPG, CHUNK), jnp.int32),
        pltpu.VMEM((KPG, CHUNK), jnp.int32),
        pltpu.VMEM((CHUNK, CPB), jnp.float32),
        pltpu.VMEM((CHUNK,), jnp.int32),
        pltpu.VMEM((REM, CPB), jnp.float32),
        pltpu.VMEM((REM,), jnp.int32),
        pltpu.VMEM_SHARED((N_SEG, CPB), jnp.float32),
        pltpu.SemaphoreType.DMA,
        pltpu.SemaphoreType.DMA,
        pltpu.SemaphoreType.DMA,
        pltpu.SemaphoreType.DMA,
        pltpu.SemaphoreType.DMA,
        pltpu.SemaphoreType.DMA,
        pltpu.SemaphoreType.DMA,
        pltpu.SemaphoreType.DMA,
        pltpu.SemaphoreType.DMA,
        pltpu.SemaphoreType.DMA,
        pltpu.SemaphoreType.DMA,
        pltpu.SemaphoreType.DMA,
        pltpu.SemaphoreType.DMA,
    ],
    compiler_params=pltpu.CompilerParams(use_tc_tiling_on_sc=False),
)
def _seg_sum_sc(feat_hbm, ids_hbm, out_hbm,
                rows0, rows1, rows2, rows3, rows4, rows5,
                idx0, idx1, idx2, idx3, idx4, idx5,
                rows_t, idx_t, rows_r, idx_r, acc,
                ld0, ld1, ld2, ld3, ld4, ld5,
                sc0, sc1, sc2, sc3, sc4, sc5, lde):
    c = lax.axis_index("c")
    s = lax.axis_index("s")
    col0 = c * CPB
    gstart = s * GPT

    rows = (rows0, rows1, rows2, rows3, rows4, rows5)
    idx = (idx0, idx1, idx2, idx3, idx4, idx5)
    ld = (ld0, ld1, ld2, ld3, ld4, ld5)
    sc = (sc0, sc1, sc2, sc3, sc4, sc5)

    def start_load(g, b):
        off = (gstart + g) * GROUP
        pltpu.async_copy(
            feat_hbm.at[pl.ds(off, GROUP), pl.ds(col0, CPB)], rows[b], ld[b])
        for k in range(KPG):
            pltpu.async_copy(ids_hbm.at[pl.ds(off + k * CHUNK, CHUNK)],
                             idx[b].at[k], ld[b])

    def wait_load(b):
        pltpu.make_async_copy(feat_hbm.at[pl.ds(0, GROUP), pl.ds(0, CPB)],
                              rows[b], ld[b]).wait()
        for k in range(KPG):
            pltpu.make_async_copy(ids_hbm.at[pl.ds(0, CHUNK)],
                                  idx[b].at[k], ld[b]).wait()

    def start_scatters(b):
        for k in range(KPG):
            pltpu.async_copy(rows[b].at[pl.ds(k * CHUNK, CHUNK)],
                             acc.at[idx[b].at[k]], sc[b], add=True)

    def wait_scatters(b):
        for k in range(KPG):
            pltpu.make_async_copy(rows[b].at[pl.ds(k * CHUNK, CHUNK)],
                                  acc.at[idx[b].at[k]], sc[b]).wait()

    zrow = jnp.zeros((16,), jnp.float32)

    def zero_body(r, carry):
        for j in range(CPB // 16):
            rows0[r, pl.ds(j * 16, 16)] = zrow
        return carry

    lax.fori_loop(0, SEG_PER_TILE, zero_body, 0)
    pltpu.sync_copy(rows0.at[pl.ds(0, SEG_PER_TILE)],
                    acc.at[pl.ds(s * SEG_PER_TILE, SEG_PER_TILE)])

    for b in range(NBUF):
        start_load(b, b)

    @pl.when(s < N_EXTRA)
    def _():
        off = EXTRA_OFF + s * CHUNK
        pltpu.async_copy(feat_hbm.at[pl.ds(off, CHUNK), pl.ds(col0, CPB)],
                         rows_t, lde)
        pltpu.async_copy(ids_hbm.at[pl.ds(off, CHUNK)], idx_t, lde)

    @pl.when(s == N_EXTRA)
    def _():
        pltpu.async_copy(feat_hbm.at[pl.ds(TAIL_OFF, CHUNK), pl.ds(col0, CPB)],
                         rows_t, lde)
        pltpu.async_copy(ids_hbm.at[pl.ds(TAIL_OFF, CHUNK)], idx_t, lde)

    @pl.when(s == NS - 1)
    def _():
        pltpu.async_copy(feat_hbm.at[pl.ds(REM_OFF, REM), pl.ds(col0, CPB)],
                         rows_r, lde)
        pltpu.async_copy(ids_hbm.at[pl.ds(REM_OFF, REM)], idx_r, lde)

    plsc.subcore_barrier()

    rounds = [list(range(r, min(r + NBUF, GPT))) for r in range(0, GPT, NBUF)]
    for rnd in rounds:
        for g in rnd:
            b = g % NBUF
            wait_load(b)
            start_scatters(b)
        for g in rnd:
            b = g % NBUF
            wait_scatters(b)
            if g + NBUF < GPT:
                start_load(g + NBUF, b)

    @pl.when(s <= N_EXTRA)
    def _():
        pltpu.make_async_copy(feat_hbm.at[pl.ds(0, CHUNK), pl.ds(0, CPB)],
                              rows_t, lde).wait()
        pltpu.make_async_copy(ids_hbm.at[pl.ds(0, CHUNK)], idx_t, lde).wait()
        pltpu.sync_copy(rows_t, acc.at[idx_t], add=True)

    @pl.when(s == NS - 1)
    def _():
        pltpu.make_async_copy(feat_hbm.at[pl.ds(0, REM), pl.ds(0, CPB)],
                              rows_r, lde).wait()
        pltpu.make_async_copy(ids_hbm.at[pl.ds(0, REM)], idx_r, lde).wait()
        pltpu.sync_copy(rows_r, acc.at[idx_r], add=True)

    plsc.subcore_barrier()
    pltpu.sync_copy(acc.at[pl.ds(s * SEG_PER_TILE, SEG_PER_TILE)],
                    out_hbm.at[pl.ds(s * SEG_PER_TILE, SEG_PER_TILE),
                               pl.ds(col0, CPB)])


def _tc_body(ids_ref, feat_ref, out_ref):
    i = pl.program_id(0)
    ids = ids_ref[...]
    seg = lax.broadcasted_iota(jnp.int32, (N_SEG, R_TC), 0)
    onehot = (seg == ids[None, :]).astype(jnp.bfloat16)
    fb = feat_ref[...].astype(jnp.bfloat16)
    part = jnp.dot(onehot, fb, preferred_element_type=jnp.float32)

    @pl.when(i == 0)
    def _():
        out_ref[...] = part

    @pl.when(i > 0)
    def _():
        out_ref[...] += part


_seg_sum_tc = pl.pallas_call(
    _tc_body,
    grid=(TC_CHUNKS,),
    in_specs=[
        pl.BlockSpec((R_TC,), lambda i: (SC_ROWS // R_TC + i,)),
        pl.BlockSpec((R_TC, N_COLS), lambda i: (SC_ROWS // R_TC + i, 0)),
    ],
    out_specs=pl.BlockSpec((N_SEG, N_COLS), lambda i: (0, 0)),
    out_shape=jax.ShapeDtypeStruct((N_SEG, N_COLS), jnp.float32),
)


def kernel(feat, segment_ids):
    ids = segment_ids.astype(jnp.int32)
    sc_part = _seg_sum_sc(feat, ids)
    tc_part = _seg_sum_tc(ids, feat)
    return sc_part + tc_part
